# Initial kernel scaffold; baseline (speedup 1.0000x reference)
import jax
import jax.numpy as jnp
from jax.experimental import pallas as pl


def kernel(x, edge_index, W1_l, W1_r, b1, W2_l, W2_r, b2):
    raise NotImplementedError("write your pallas kernel here")


# speedup vs baseline: 4.1193x; 4.1193x over previous
import jax
import jax.numpy as jnp
from jax import lax
from jax.experimental import pallas as pl
from jax.experimental.pallas import tpu as pltpu
from jax.experimental.pallas import tpu_sc as plsc

N_SUB = 16
N_CORES = 2
CHUNK = 128
DH = 128
RB = 1000


def _make_sc_agg(n_nodes, ep, compute_deg):
    kt = ep // (N_SUB * CHUNK)
    kth = kt // 2
    np_rows = -(-(n_nodes + 1) // (N_SUB * 8)) * (N_SUB * 8)
    zrows = np_rows // N_SUB
    orows = (n_nodes // N_SUB) & ~7
    oextra = n_nodes - orows * N_SUB
    assert oextra % 8 == 0 and oextra <= CHUNK

    mesh = plsc.VectorSubcoreMesh(core_axis_name="c", subcore_axis_name="s",
                                  num_cores=N_CORES, num_subcores=N_SUB)
    out_types = [jax.ShapeDtypeStruct((2 * n_nodes, DH), jnp.float32)]
    scratch = [
        pltpu.VMEM((2, CHUNK), jnp.int32),
        pltpu.VMEM((CHUNK, DH), jnp.float32),
        pltpu.VMEM_SHARED((np_rows, DH), jnp.float32),
    ]
    dstg_len = -(-max(zrows, orows + 8) // 16) * 16
    if compute_deg:
        out_types.append(jax.ShapeDtypeStruct((2 * n_nodes,), jnp.float32))
        scratch += [
            pltpu.VMEM((CHUNK,), jnp.float32),
            pltpu.VMEM((dstg_len,), jnp.float32),
            pltpu.VMEM_SHARED((np_rows,), jnp.float32),
        ]

    def body(x_hbm, eidx_hbm, *rest):
        if compute_deg:
            agg_hbm, deg_hbm, idx, rows, acc, ones, dstg, dacc = rest
        else:
            agg_hbm, idx, rows, acc = rest
        stage = rows
        c = lax.axis_index("c")
        s = lax.axis_index("s")
        coff = c * n_nodes

        @pl.loop(0, CHUNK)
        def _(i):
            @pl.loop(0, DH, step=16)
            def _(j):
                stage[i, pl.ds(j, 16)] = jnp.zeros((16,), jnp.float32)

        if compute_deg:
            @pl.loop(0, CHUNK, step=16)
            def _(i):
                ones[pl.ds(i, 16)] = jnp.ones((16,), jnp.float32)

            @pl.loop(0, dstg_len, step=16)
            def _(i):
                dstg[pl.ds(i, 16)] = jnp.zeros((16,), jnp.float32)

        zbase = s * zrows
        znf = zrows // CHUNK
        zrem = zrows - znf * CHUNK

        @pl.loop(0, znf)
        def _(i):
            pltpu.sync_copy(stage, acc.at[pl.ds(zbase + i * CHUNK, CHUNK)])
        if zrem:
            pltpu.sync_copy(stage.at[pl.ds(0, zrem)],
                            acc.at[pl.ds(zbase + znf * CHUNK, zrem)])
        if compute_deg:
            pltpu.sync_copy(dstg.at[pl.ds(0, zrows)],
                            dacc.at[pl.ds(zbase, zrows)])
        plsc.subcore_barrier()

        ebase = s * kt

        @pl.loop(0, kt)
        def _(k):
            pltpu.sync_copy(eidx_hbm.at[ebase + k], idx)

            @pl.loop(0, CHUNK, step=16)
            def _(j):
                idx[0, pl.ds(j, 16)] = idx[0, pl.ds(j, 16)] + coff

            pltpu.sync_copy(x_hbm.at[idx.at[0]], rows)
            pltpu.sync_copy(rows, acc.at[idx.at[1]], add=True)
            if compute_deg:
                @pl.when(jnp.logical_xor(k >= kth, c == 0))
                def _():
                    pltpu.sync_copy(ones, dacc.at[idx.at[1]], add=True)

        plsc.subcore_barrier()

        obase = s * orows
        onf = orows // CHUNK
        orem = orows - onf * CHUNK

        def copy_rows(r0, cnt, src_sh, stg, out_hbm):
            pltpu.sync_copy(src_sh.at[pl.ds(r0, cnt)], stg.at[pl.ds(0, cnt)])
            pltpu.sync_copy(stg.at[pl.ds(0, cnt)],
                            out_hbm.at[pl.ds(coff + r0, cnt)])

        @pl.loop(0, onf)
        def _(i):
            copy_rows(obase + i * CHUNK, CHUNK, acc, stage, agg_hbm)
        if orem:
            copy_rows(obase + onf * CHUNK, orem, acc, stage, agg_hbm)
        if oextra:
            @pl.when(s == N_SUB - 1)
            def _():
                copy_rows(N_SUB * orows, oextra, acc, stage, agg_hbm)
        if compute_deg:
            copy_rows(obase, orows, dacc, dstg, deg_hbm)
            if oextra:
                @pl.when(s == N_SUB - 1)
                def _():
                    copy_rows(N_SUB * orows, oextra, dacc, dstg, deg_hbm)

    out_type = out_types if compute_deg else out_types[0]
    return pl.kernel(body, out_type=out_type, mesh=mesh, scratch_types=scratch)


def _make_tc_dense(n_nodes, split_out):
    grid = (n_nodes // RB,)

    def body(agg_ref, deg_ref, xs_ref, wl_ref, wr_ref, b_ref, out_ref):
        deg = deg_ref[0, 0, 0, :] + deg_ref[1, 0, 0, :]
        inv = 1.0 / jnp.maximum(deg, 1.0)
        a = jnp.concatenate([agg_ref[0], agg_ref[1]], axis=1) * inv[:, None]
        xf = jnp.concatenate([xs_ref[0], xs_ref[1]], axis=1)
        h = (jnp.dot(a, wl_ref[...], preferred_element_type=jnp.float32)
             + b_ref[...]
             + jnp.dot(xf, wr_ref[...], preferred_element_type=jnp.float32))
        h = jnp.maximum(h, 0.0)
        if split_out:
            out_ref[0] = h[:, :DH]
            out_ref[1] = h[:, DH:]
        else:
            out_ref[...] = h

    in_specs = [
        pl.BlockSpec((2, RB, DH), lambda i: (0, i, 0)),
        pl.BlockSpec((2, 1, 1, RB), lambda i: (0, i, 0, 0)),
        pl.BlockSpec((2, RB, DH), lambda i: (0, i, 0)),
        pl.BlockSpec((2 * DH, 2 * DH), lambda i: (0, 0)),
        pl.BlockSpec((2 * DH, 2 * DH), lambda i: (0, 0)),
        pl.BlockSpec((1, 2 * DH), lambda i: (0, 0)),
    ]
    if split_out:
        out_spec = pl.BlockSpec((2, RB, DH), lambda i: (0, i, 0))
        out_shape = jax.ShapeDtypeStruct((2, n_nodes, DH), jnp.float32)
    else:
        out_spec = pl.BlockSpec((RB, 2 * DH), lambda i: (i, 0))
        out_shape = jax.ShapeDtypeStruct((n_nodes, 2 * DH), jnp.float32)
    return pl.pallas_call(body, grid=grid, in_specs=in_specs,
                          out_specs=out_spec, out_shape=out_shape)


def kernel(x, edge_index, W1_l, W1_r, b1, W2_l, W2_r, b2):
    n, d = x.shape
    e = edge_index.shape[1]
    assert d == 2 * DH

    ep = -(-e // (N_SUB * CHUNK)) * (N_SUB * CHUNK)
    src = edge_index[0]
    dst = edge_index[1]
    if ep != e:
        pad = ep - e
        src = jnp.concatenate([src, jnp.zeros((pad,), jnp.int32)])
        dst = jnp.concatenate([dst, jnp.full((pad,), n, jnp.int32)])
    eidx = jnp.stack([src.reshape(-1, CHUNK), dst.reshape(-1, CHUNK)], axis=1)

    x_split = jnp.stack([x[:, :DH], x[:, DH:]])
    x_flat = x_split.reshape(2 * n, DH)

    sc1 = _make_sc_agg(n, ep, True)
    agg1, deg = sc1(x_flat, eidx)

    deg4 = deg.reshape(2, n // RB, 1, RB)
    tc1 = _make_tc_dense(n, True)
    h1_split = tc1(agg1.reshape(2, n, DH), deg4, x_split,
                   W1_l.T, W1_r.T, b1.reshape(1, 2 * DH))

    sc2 = _make_sc_agg(n, ep, False)
    agg2 = sc2(h1_split.reshape(2 * n, DH), eidx)

    tc2 = _make_tc_dense(n, False)
    out = tc2(agg2.reshape(2, n, DH), deg4, h1_split,
              W2_l.T, W2_r.T, b2.reshape(1, 2 * DH))
    return out


# speedup vs baseline: 4.8637x; 1.1807x over previous
import jax
import jax.numpy as jnp
from jax import lax
from jax.experimental import pallas as pl
from jax.experimental.pallas import tpu as pltpu
from jax.experimental.pallas import tpu_sc as plsc

N_SUB = 16
N_CORES = 2
CHUNK = 64
DH = 128
RB = 1000


def _make_sc_agg(n_nodes, ep, compute_deg):
    kt = ep // (N_SUB * CHUNK)
    kth = kt // 2
    np_rows = -(-(n_nodes + 1) // (N_SUB * 8)) * (N_SUB * 8)
    zrows = np_rows // N_SUB
    orows = (n_nodes // N_SUB) & ~7
    oextra = n_nodes - orows * N_SUB
    assert oextra % 8 == 0 and oextra <= CHUNK

    mesh = plsc.VectorSubcoreMesh(core_axis_name="c", subcore_axis_name="s",
                                  num_cores=N_CORES, num_subcores=N_SUB)
    assert kt % 2 == 0
    out_types = [jax.ShapeDtypeStruct((2 * n_nodes, DH), jnp.float32)]
    scratch = [
        pltpu.VMEM((2, CHUNK), jnp.int32),
        pltpu.VMEM((2, CHUNK), jnp.int32),
        pltpu.VMEM((CHUNK, DH), jnp.float32),
        pltpu.VMEM((CHUNK, DH), jnp.float32),
        pltpu.VMEM_SHARED((np_rows, DH), jnp.float32),
        pltpu.SemaphoreType.DMA,
        pltpu.SemaphoreType.DMA,
    ]
    dstg_len = -(-max(zrows, orows + 8) // 16) * 16
    if compute_deg:
        out_types.append(jax.ShapeDtypeStruct((2 * n_nodes,), jnp.float32))
        scratch += [
            pltpu.VMEM((CHUNK,), jnp.float32),
            pltpu.VMEM((dstg_len,), jnp.float32),
            pltpu.VMEM_SHARED((np_rows,), jnp.float32),
        ]

    def body(x_hbm, eidx_hbm, *rest):
        if compute_deg:
            (agg_hbm, deg_hbm, idx0, idx1, rows0, rows1, acc, gsem0, gsem1,
             ones, dstg, dacc) = rest
        else:
            agg_hbm, idx0, idx1, rows0, rows1, acc, gsem0, gsem1 = rest
        stage = rows0
        c = lax.axis_index("c")
        s = lax.axis_index("s")
        coff = c * n_nodes

        @pl.loop(0, CHUNK)
        def _(i):
            @pl.loop(0, DH, step=16)
            def _(j):
                stage[i, pl.ds(j, 16)] = jnp.zeros((16,), jnp.float32)

        if compute_deg:
            @pl.loop(0, CHUNK, step=16)
            def _(i):
                ones[pl.ds(i, 16)] = jnp.ones((16,), jnp.float32)

            @pl.loop(0, dstg_len, step=16)
            def _(i):
                dstg[pl.ds(i, 16)] = jnp.zeros((16,), jnp.float32)

        zbase = s * zrows
        znf = zrows // CHUNK
        zrem = zrows - znf * CHUNK

        @pl.loop(0, znf)
        def _(i):
            pltpu.sync_copy(stage, acc.at[pl.ds(zbase + i * CHUNK, CHUNK)])
        if zrem:
            pltpu.sync_copy(stage.at[pl.ds(0, zrem)],
                            acc.at[pl.ds(zbase + znf * CHUNK, zrem)])
        if compute_deg:
            pltpu.sync_copy(dstg.at[pl.ds(0, zrows)],
                            dacc.at[pl.ds(zbase, zrows)])
        plsc.subcore_barrier()

        ebase = s * kt

        def load_adjust(k, idx):
            pltpu.sync_copy(eidx_hbm.at[ebase + k], idx)

            @pl.loop(0, CHUNK, step=16)
            def _(j):
                idx[0, pl.ds(j, 16)] = idx[0, pl.ds(j, 16)] + coff

            if compute_deg:
                @pl.when(jnp.logical_xor(k >= kth, c == 0))
                def _():
                    pltpu.sync_copy(ones, dacc.at[idx.at[1]], add=True)

        def start_gather(idx, rows, sem):
            pltpu.async_copy(x_hbm.at[idx.at[0]], rows, sem)

        def wait_gather(idx, rows, sem):
            pltpu.make_async_copy(x_hbm.at[idx.at[0]], rows, sem).wait()

        def scatter(idx, rows):
            pltpu.sync_copy(rows, acc.at[idx.at[1]], add=True)

        load_adjust(0, idx0)
        start_gather(idx0, rows0, gsem0)
        load_adjust(1, idx1)
        start_gather(idx1, rows1, gsem1)

        @pl.loop(0, kt // 2 - 1)
        def _(p):
            k0 = 2 * p
            wait_gather(idx0, rows0, gsem0)
            scatter(idx0, rows0)
            load_adjust(k0 + 2, idx0)
            start_gather(idx0, rows0, gsem0)
            wait_gather(idx1, rows1, gsem1)
            scatter(idx1, rows1)
            load_adjust(k0 + 3, idx1)
            start_gather(idx1, rows1, gsem1)

        wait_gather(idx0, rows0, gsem0)
        scatter(idx0, rows0)
        wait_gather(idx1, rows1, gsem1)
        scatter(idx1, rows1)

        plsc.subcore_barrier()

        obase = s * orows
        onf = orows // CHUNK
        orem = orows - onf * CHUNK

        def copy_rows(r0, cnt, src_sh, stg, out_hbm):
            pltpu.sync_copy(src_sh.at[pl.ds(r0, cnt)], stg.at[pl.ds(0, cnt)])
            pltpu.sync_copy(stg.at[pl.ds(0, cnt)],
                            out_hbm.at[pl.ds(coff + r0, cnt)])

        @pl.loop(0, onf)
        def _(i):
            copy_rows(obase + i * CHUNK, CHUNK, acc, stage, agg_hbm)
        if orem:
            copy_rows(obase + onf * CHUNK, orem, acc, stage, agg_hbm)
        if oextra:
            @pl.when(s == N_SUB - 1)
            def _():
                copy_rows(N_SUB * orows, oextra, acc, stage, agg_hbm)
        if compute_deg:
            copy_rows(obase, orows, dacc, dstg, deg_hbm)
            if oextra:
                @pl.when(s == N_SUB - 1)
                def _():
                    copy_rows(N_SUB * orows, oextra, dacc, dstg, deg_hbm)

    out_type = out_types if compute_deg else out_types[0]
    return pl.kernel(body, out_type=out_type, mesh=mesh, scratch_types=scratch)


def _make_tc_dense(n_nodes, split_out):
    grid = (n_nodes // RB,)

    def body(agg_ref, deg_ref, xs_ref, wl_ref, wr_ref, b_ref, out_ref):
        deg = deg_ref[0, 0, 0, :] + deg_ref[1, 0, 0, :]
        inv = 1.0 / jnp.maximum(deg, 1.0)
        a = jnp.concatenate([agg_ref[0], agg_ref[1]], axis=1) * inv[:, None]
        xf = jnp.concatenate([xs_ref[0], xs_ref[1]], axis=1)
        h = (jnp.dot(a, wl_ref[...], preferred_element_type=jnp.float32)
             + b_ref[...]
             + jnp.dot(xf, wr_ref[...], preferred_element_type=jnp.float32))
        h = jnp.maximum(h, 0.0)
        if split_out:
            out_ref[0] = h[:, :DH]
            out_ref[1] = h[:, DH:]
        else:
            out_ref[...] = h

    in_specs = [
        pl.BlockSpec((2, RB, DH), lambda i: (0, i, 0)),
        pl.BlockSpec((2, 1, 1, RB), lambda i: (0, i, 0, 0)),
        pl.BlockSpec((2, RB, DH), lambda i: (0, i, 0)),
        pl.BlockSpec((2 * DH, 2 * DH), lambda i: (0, 0)),
        pl.BlockSpec((2 * DH, 2 * DH), lambda i: (0, 0)),
        pl.BlockSpec((1, 2 * DH), lambda i: (0, 0)),
    ]
    if split_out:
        out_spec = pl.BlockSpec((2, RB, DH), lambda i: (0, i, 0))
        out_shape = jax.ShapeDtypeStruct((2, n_nodes, DH), jnp.float32)
    else:
        out_spec = pl.BlockSpec((RB, 2 * DH), lambda i: (i, 0))
        out_shape = jax.ShapeDtypeStruct((n_nodes, 2 * DH), jnp.float32)
    return pl.pallas_call(body, grid=grid, in_specs=in_specs,
                          out_specs=out_spec, out_shape=out_shape)


def kernel(x, edge_index, W1_l, W1_r, b1, W2_l, W2_r, b2):
    n, d = x.shape
    e = edge_index.shape[1]
    assert d == 2 * DH

    ep = -(-e // (2 * N_SUB * CHUNK)) * (2 * N_SUB * CHUNK)
    src = edge_index[0]
    dst = edge_index[1]
    if ep != e:
        pad = ep - e
        src = jnp.concatenate([src, jnp.zeros((pad,), jnp.int32)])
        dst = jnp.concatenate([dst, jnp.full((pad,), n, jnp.int32)])
    eidx = jnp.stack([src.reshape(-1, CHUNK), dst.reshape(-1, CHUNK)], axis=1)

    x_split = jnp.stack([x[:, :DH], x[:, DH:]])
    x_flat = x_split.reshape(2 * n, DH)

    sc1 = _make_sc_agg(n, ep, True)
    agg1, deg = sc1(x_flat, eidx)

    deg4 = deg.reshape(2, n // RB, 1, RB)
    tc1 = _make_tc_dense(n, True)
    h1_split = tc1(agg1.reshape(2, n, DH), deg4, x_split,
                   W1_l.T, W1_r.T, b1.reshape(1, 2 * DH))

    sc2 = _make_sc_agg(n, ep, False)
    agg2 = sc2(h1_split.reshape(2 * n, DH), eidx)

    tc2 = _make_tc_dense(n, False)
    out = tc2(agg2.reshape(2, n, DH), deg4, h1_split,
              W2_l.T, W2_r.T, b2.reshape(1, 2 * DH))
    return out
